# baseline (speedup 1.0000x reference)
import jax
import jax.numpy as jnp
from jax import lax
from jax.experimental import pallas as pl
from jax.experimental.pallas import tpu as pltpu
from jax.experimental.pallas import tpu_sc as plsc

N_NODES = 10000
N_EDGES = 320000
D_FEAT = 128

NC = 2
NS = 16
NW = NC * NS
L = 16

EW = N_EDGES // NW
C = 80
NCHUNK = EW // C
NGROUP = C // L


def _tec_body(z_hbm, src_hbm, dst_hbm, out_hbm, z_sh,
              idx_s0, idx_d0, idx_s1, idx_d1,
              rows_s0, rows_d0, rows_s1, rows_d1,
              out0, out1,
              sem_i0, sem_i1, sem_r0, sem_r1, sem_o0, sem_o1):
    sid = lax.axis_index("s")
    wid = sid * NC + lax.axis_index("c")
    base = wid * EW

    idx_bufs = ((idx_s0, idx_d0), (idx_s1, idx_d1))
    rows_bufs = ((rows_s0, rows_d0), (rows_s1, rows_d1))
    outs = (out0, out1)
    sem_idx = (sem_i0, sem_i1)
    sem_rows = (sem_r0, sem_r1)
    sem_out = (sem_o0, sem_o1)
    iota = lax.iota(jnp.int32, L)

    def issue_idx(g, b):
        off = base + g * C
        pltpu.async_copy(src_hbm.at[pl.ds(off, C)], idx_bufs[b][0], sem_idx[b])
        pltpu.async_copy(dst_hbm.at[pl.ds(off, C)], idx_bufs[b][1], sem_idx[b])

    def wait_idx(b):
        pltpu.make_async_copy(src_hbm.at[pl.ds(0, C)], idx_bufs[b][0],
                              sem_idx[b]).wait()
        pltpu.make_async_copy(dst_hbm.at[pl.ds(0, C)], idx_bufs[b][1],
                              sem_idx[b]).wait()

    def issue_gather(b):
        pltpu.async_copy(z_sh.at[idx_bufs[b][0]], rows_bufs[b][0], sem_rows[b])
        pltpu.async_copy(z_sh.at[idx_bufs[b][1]], rows_bufs[b][1], sem_rows[b])

    def wait_gather(b):
        pltpu.make_async_copy(z_hbm.at[pl.ds(0, C)], rows_bufs[b][0],
                              sem_rows[b]).wait()
        pltpu.make_async_copy(z_hbm.at[pl.ds(0, C)], rows_bufs[b][1],
                              sem_rows[b]).wait()

    def issue_out(g, b):
        pltpu.async_copy(outs[b], out_hbm.at[pl.ds(base + g * C, C)],
                         sem_out[b])

    def wait_out(b):
        pltpu.make_async_copy(outs[b], out_hbm.at[pl.ds(0, C)],
                              sem_out[b]).wait()

    def compute(b):
        src_rows, dst_rows = rows_bufs[b]
        out_buf = outs[b]

        def group_body(gi, carry2):
            gbase = gi * L

            def edge_body(j, acc):
                e = gbase + j
                p = src_rows[e, pl.ds(0, L)] * dst_rows[e, pl.ds(0, L)]
                for kk in range(1, D_FEAT // L):
                    p = p + (src_rows[e, pl.ds(kk * L, L)]
                             * dst_rows[e, pl.ds(kk * L, L)])
                return jnp.where(iota == j, jnp.sum(p), acc)

            acc = lax.fori_loop(0, L, edge_body,
                                jnp.zeros((L,), jnp.float32), unroll=8)
            out = 1.0 / (1.0 + jnp.exp(-acc))
            out_buf[pl.ds(gbase, L)] = out
            return carry2

        lax.fori_loop(0, NGROUP, group_body, 0, unroll=False)

    def sub_iter(g, b):
        wait_gather(b)

        @pl.when(g + 2 < NCHUNK)
        def _():
            issue_idx(g + 2, b)

        @pl.when(g + 1 < NCHUNK)
        def _():
            wait_idx(1 - b)
            issue_gather(1 - b)

        @pl.when(g >= 2)
        def _():
            wait_out(b)

        compute(b)
        issue_out(g, b)

    @pl.when(sid < NS - 1)
    def _():
        pltpu.sync_copy(z_hbm.at[pl.ds(sid * 632, 632)],
                        z_sh.at[pl.ds(sid * 632, 632)])

    @pl.when(sid == NS - 1)
    def _():
        pltpu.sync_copy(z_hbm.at[pl.ds(9480, 520)],
                        z_sh.at[pl.ds(9480, 520)])

    issue_idx(0, 0)
    issue_idx(1, 1)
    plsc.subcore_barrier()
    wait_idx(0)
    issue_gather(0)

    def pair_body(i, carry):
        sub_iter(2 * i, 0)
        sub_iter(2 * i + 1, 1)
        return carry

    lax.fori_loop(0, NCHUNK // 2, pair_body, 0, unroll=False)
    sub_iter(NCHUNK - 1, 0)

    wait_out(1)
    wait_out(0)


@jax.jit
def _gae_decode(z, src, dst):
    mesh = plsc.VectorSubcoreMesh(core_axis_name="c", subcore_axis_name="s")
    k = pl.kernel(
        _tec_body,
        out_type=jax.ShapeDtypeStruct((N_EDGES,), jnp.float32),
        mesh=mesh,
        compiler_params=pltpu.CompilerParams(needs_layout_passes=False),
        scratch_types=[
            pltpu.VMEM_SHARED((N_NODES, D_FEAT), jnp.float32),
            pltpu.VMEM((C,), jnp.int32),
            pltpu.VMEM((C,), jnp.int32),
            pltpu.VMEM((C,), jnp.int32),
            pltpu.VMEM((C,), jnp.int32),
            pltpu.VMEM((C, D_FEAT), jnp.float32),
            pltpu.VMEM((C, D_FEAT), jnp.float32),
            pltpu.VMEM((C, D_FEAT), jnp.float32),
            pltpu.VMEM((C, D_FEAT), jnp.float32),
            pltpu.VMEM((C,), jnp.float32),
            pltpu.VMEM((C,), jnp.float32),
            pltpu.SemaphoreType.DMA,
            pltpu.SemaphoreType.DMA,
            pltpu.SemaphoreType.DMA,
            pltpu.SemaphoreType.DMA,
            pltpu.SemaphoreType.DMA,
            pltpu.SemaphoreType.DMA,
        ],
    )
    return k(z, src, dst)


def kernel(z, edge_index):
    return _gae_decode(z, edge_index[0], edge_index[1])


# speedup vs baseline: 1.0528x; 1.0528x over previous
import jax
import jax.numpy as jnp
from jax import lax
from jax.experimental import pallas as pl
from jax.experimental.pallas import tpu as pltpu
from jax.experimental.pallas import tpu_sc as plsc

N_NODES = 10000
N_EDGES = 320000
D_FEAT = 128

NC = 2
NS = 16
NW = NC * NS
L = 16

EW = N_EDGES // NW
C = 80
NCHUNK = EW // C
NGROUP = C // L


def _tec_body(z_hbm, idx_hbm, out_hbm, z_sh,
              idx0, idx1, rows0, rows1, out0, out1,
              sem_i0, sem_i1, sem_r0, sem_r1, sem_o0, sem_o1):
    sid = lax.axis_index("s")
    wid = sid * NC + lax.axis_index("c")
    base = wid * EW

    idx_bufs = (idx0, idx1)
    rows_bufs = (rows0, rows1)
    outs = (out0, out1)
    sem_idx = (sem_i0, sem_i1)
    sem_rows = (sem_r0, sem_r1)
    sem_out = (sem_o0, sem_o1)
    iota = lax.iota(jnp.int32, L)

    def issue_idx(g, b):
        off = (wid * NCHUNK + g) * 2 * C
        pltpu.async_copy(idx_hbm.at[pl.ds(off, 2 * C)], idx_bufs[b],
                         sem_idx[b])

    def wait_idx(b):
        pltpu.make_async_copy(idx_hbm.at[pl.ds(0, 2 * C)], idx_bufs[b],
                              sem_idx[b]).wait()

    def issue_gather(b):
        pltpu.async_copy(z_sh.at[idx_bufs[b]], rows_bufs[b], sem_rows[b])

    def wait_gather(b):
        pltpu.make_async_copy(z_hbm.at[pl.ds(0, 2 * C)], rows_bufs[b],
                              sem_rows[b]).wait()

    def issue_out(g, b):
        pltpu.async_copy(outs[b], out_hbm.at[pl.ds(base + g * C, C)],
                         sem_out[b])

    def wait_out(b):
        pltpu.make_async_copy(outs[b], out_hbm.at[pl.ds(0, C)],
                              sem_out[b]).wait()

    def compute(b):
        rows = rows_bufs[b]
        out_buf = outs[b]

        def group_body(gi, carry2):
            gbase = gi * L

            def edge_body(j, acc):
                e = gbase + j
                p = rows[e, pl.ds(0, L)] * rows[C + e, pl.ds(0, L)]
                for kk in range(1, D_FEAT // L):
                    p = p + (rows[e, pl.ds(kk * L, L)]
                             * rows[C + e, pl.ds(kk * L, L)])
                return jnp.where(iota == j, jnp.sum(p), acc)

            acc = lax.fori_loop(0, L, edge_body,
                                jnp.zeros((L,), jnp.float32), unroll=4)
            out = 1.0 / (1.0 + jnp.exp(-acc))
            out_buf[pl.ds(gbase, L)] = out
            return carry2

        lax.fori_loop(0, NGROUP, group_body, 0, unroll=False)

    def sub_iter(g, b):
        wait_gather(b)

        @pl.when(g + 2 < NCHUNK)
        def _():
            issue_idx(g + 2, b)

        @pl.when(g + 1 < NCHUNK)
        def _():
            wait_idx(1 - b)
            issue_gather(1 - b)

        @pl.when(g >= 2)
        def _():
            wait_out(b)

        compute(b)
        issue_out(g, b)

    @pl.when(sid < NS - 1)
    def _():
        pltpu.sync_copy(z_hbm.at[pl.ds(sid * 632, 632)],
                        z_sh.at[pl.ds(sid * 632, 632)])

    @pl.when(sid == NS - 1)
    def _():
        pltpu.sync_copy(z_hbm.at[pl.ds(9480, 520)],
                        z_sh.at[pl.ds(9480, 520)])

    issue_idx(0, 0)
    issue_idx(1, 1)
    plsc.subcore_barrier()
    wait_idx(0)
    issue_gather(0)

    def pair_body(i, carry):
        sub_iter(2 * i, 0)
        sub_iter(2 * i + 1, 1)
        return carry

    lax.fori_loop(0, NCHUNK // 2, pair_body, 0, unroll=False)
    sub_iter(NCHUNK - 1, 0)

    wait_out(1)
    wait_out(0)


@jax.jit
def _gae_decode(z, idx_cat):
    mesh = plsc.VectorSubcoreMesh(core_axis_name="c", subcore_axis_name="s")
    k = pl.kernel(
        _tec_body,
        out_type=jax.ShapeDtypeStruct((N_EDGES,), jnp.float32),
        mesh=mesh,
        compiler_params=pltpu.CompilerParams(needs_layout_passes=False),
        scratch_types=[
            pltpu.VMEM_SHARED((N_NODES, D_FEAT), jnp.float32),
            pltpu.VMEM((2 * C,), jnp.int32),
            pltpu.VMEM((2 * C,), jnp.int32),
            pltpu.VMEM((2 * C, D_FEAT), jnp.float32),
            pltpu.VMEM((2 * C, D_FEAT), jnp.float32),
            pltpu.VMEM((C,), jnp.float32),
            pltpu.VMEM((C,), jnp.float32),
            pltpu.SemaphoreType.DMA,
            pltpu.SemaphoreType.DMA,
            pltpu.SemaphoreType.DMA,
            pltpu.SemaphoreType.DMA,
            pltpu.SemaphoreType.DMA,
            pltpu.SemaphoreType.DMA,
        ],
    )
    return k(z, idx_cat)


def kernel(z, edge_index):
    src = edge_index[0].reshape(NW, NCHUNK, 1, C)
    dst = edge_index[1].reshape(NW, NCHUNK, 1, C)
    idx_cat = jnp.concatenate([src, dst], axis=2).reshape(-1)
    return _gae_decode(z, idx_cat)


# speedup vs baseline: 1.2246x; 1.1632x over previous
import jax
import jax.numpy as jnp
from jax import lax
from jax.experimental import pallas as pl
from jax.experimental.pallas import tpu as pltpu
from jax.experimental.pallas import tpu_sc as plsc

N_NODES = 10000
N_EDGES = 320000
D_FEAT = 128

NC = 2
NS = 16
NW = NC * NS
L = 16

EW = N_EDGES // NW
C = 80
NCHUNK = EW // C
NGROUP = C // L


def _tec_body(z_hbm, ei_hbm, out_hbm, z_sh,
              idx_s0, idx_d0, idx_s1, idx_d1,
              rows_s0, rows_d0, rows_s1, rows_d1,
              out0, out1,
              sem_i0, sem_i1, sem_r0, sem_r1, sem_o0, sem_o1):
    sid = lax.axis_index("s")
    wid = sid * NC + lax.axis_index("c")
    base = wid * EW

    idx_bufs = ((idx_s0, idx_d0), (idx_s1, idx_d1))
    rows_bufs = ((rows_s0, rows_d0), (rows_s1, rows_d1))
    outs = (out0, out1)
    sem_idx = (sem_i0, sem_i1)
    sem_rows = (sem_r0, sem_r1)
    sem_out = (sem_o0, sem_o1)
    iota = lax.iota(jnp.int32, L)

    def issue_idx(g, b):
        off = base + g * C
        pltpu.async_copy(ei_hbm.at[pl.ds(off, C)], idx_bufs[b][0],
                         sem_idx[b])
        pltpu.async_copy(ei_hbm.at[pl.ds(N_EDGES + off, C)], idx_bufs[b][1],
                         sem_idx[b])

    def wait_idx(b):
        pltpu.make_async_copy(ei_hbm.at[pl.ds(0, C)], idx_bufs[b][0],
                              sem_idx[b]).wait()
        pltpu.make_async_copy(ei_hbm.at[pl.ds(0, C)], idx_bufs[b][1],
                              sem_idx[b]).wait()

    def issue_gather(b):
        pltpu.async_copy(z_sh.at[idx_bufs[b][0]], rows_bufs[b][0], sem_rows[b])
        pltpu.async_copy(z_sh.at[idx_bufs[b][1]], rows_bufs[b][1], sem_rows[b])

    def wait_gather(b):
        pltpu.make_async_copy(z_hbm.at[pl.ds(0, C)], rows_bufs[b][0],
                              sem_rows[b]).wait()
        pltpu.make_async_copy(z_hbm.at[pl.ds(0, C)], rows_bufs[b][1],
                              sem_rows[b]).wait()

    def issue_out(g, b):
        pltpu.async_copy(outs[b], out_hbm.at[pl.ds(base + g * C, C)],
                         sem_out[b])

    def wait_out(b):
        pltpu.make_async_copy(outs[b], out_hbm.at[pl.ds(0, C)],
                              sem_out[b]).wait()

    def compute(b):
        src_rows, dst_rows = rows_bufs[b]
        out_buf = outs[b]

        def group_body(gi, carry2):
            gbase = gi * L

            def edge_body(j, acc):
                e = gbase + j
                p = src_rows[e, pl.ds(0, L)] * dst_rows[e, pl.ds(0, L)]
                for kk in range(1, D_FEAT // L):
                    p = p + (src_rows[e, pl.ds(kk * L, L)]
                             * dst_rows[e, pl.ds(kk * L, L)])
                return jnp.where(iota == j, jnp.sum(p), acc)

            acc = lax.fori_loop(0, L, edge_body,
                                jnp.zeros((L,), jnp.float32), unroll=4)
            out = 1.0 / (1.0 + jnp.exp(-acc))
            out_buf[pl.ds(gbase, L)] = out
            return carry2

        lax.fori_loop(0, NGROUP, group_body, 0, unroll=False)

    def sub_iter(g, b):
        wait_gather(b)

        @pl.when(g + 2 < NCHUNK)
        def _():
            issue_idx(g + 2, b)

        @pl.when(g + 1 < NCHUNK)
        def _():
            wait_idx(1 - b)
            issue_gather(1 - b)

        @pl.when(g >= 2)
        def _():
            wait_out(b)

        compute(b)
        issue_out(g, b)

    @pl.when(sid < NS - 1)
    def _():
        pltpu.sync_copy(z_hbm.at[pl.ds(sid * 632, 632)],
                        z_sh.at[pl.ds(sid * 632, 632)])

    @pl.when(sid == NS - 1)
    def _():
        pltpu.sync_copy(z_hbm.at[pl.ds(9480, 520)],
                        z_sh.at[pl.ds(9480, 520)])

    issue_idx(0, 0)
    issue_idx(1, 1)
    plsc.subcore_barrier()
    wait_idx(0)
    issue_gather(0)

    def pair_body(i, carry):
        sub_iter(2 * i, 0)
        sub_iter(2 * i + 1, 1)
        return carry

    lax.fori_loop(0, NCHUNK // 2, pair_body, 0, unroll=False)
    sub_iter(NCHUNK - 1, 0)

    wait_out(1)
    wait_out(0)


@jax.jit
def _gae_decode(z, edge_index):
    mesh = plsc.VectorSubcoreMesh(core_axis_name="c", subcore_axis_name="s")
    k = pl.kernel(
        _tec_body,
        out_type=jax.ShapeDtypeStruct((N_EDGES,), jnp.float32),
        mesh=mesh,
        compiler_params=pltpu.CompilerParams(needs_layout_passes=False),
        scratch_types=[
            pltpu.VMEM_SHARED((N_NODES, D_FEAT), jnp.float32),
            pltpu.VMEM((C,), jnp.int32),
            pltpu.VMEM((C,), jnp.int32),
            pltpu.VMEM((C,), jnp.int32),
            pltpu.VMEM((C,), jnp.int32),
            pltpu.VMEM((C, D_FEAT), jnp.float32),
            pltpu.VMEM((C, D_FEAT), jnp.float32),
            pltpu.VMEM((C, D_FEAT), jnp.float32),
            pltpu.VMEM((C, D_FEAT), jnp.float32),
            pltpu.VMEM((C,), jnp.float32),
            pltpu.VMEM((C,), jnp.float32),
            pltpu.SemaphoreType.DMA,
            pltpu.SemaphoreType.DMA,
            pltpu.SemaphoreType.DMA,
            pltpu.SemaphoreType.DMA,
            pltpu.SemaphoreType.DMA,
            pltpu.SemaphoreType.DMA,
        ],
    )
    return k(z, edge_index)


def kernel(z, edge_index):
    return _gae_decode(z, edge_index.reshape(-1))


# speedup vs baseline: 1.2254x; 1.0006x over previous
import jax
import jax.numpy as jnp
from jax import lax
from jax.experimental import pallas as pl
from jax.experimental.pallas import tpu as pltpu
from jax.experimental.pallas import tpu_sc as plsc

N_NODES = 10000
N_EDGES = 320000
D_FEAT = 128

NC = 2
NS = 16
NW = NC * NS
L = 16

EW = N_EDGES // NW
C = 80
NCHUNK = EW // C
NGROUP = C // L


def _tec_body(z_hbm, ei_hbm, out_hbm, z_sh,
              idx_s0, idx_d0, idx_s1, idx_d1,
              rows_s0, rows_d0, rows_s1, rows_d1,
              out0, out1,
              sem_i0, sem_i1, sem_r0, sem_r1, sem_o0, sem_o1):
    sid = lax.axis_index("s")
    wid = sid * NC + lax.axis_index("c")
    base = wid * EW

    idx_bufs = ((idx_s0, idx_d0), (idx_s1, idx_d1))
    rows_bufs = ((rows_s0, rows_d0), (rows_s1, rows_d1))
    outs = (out0, out1)
    sem_idx = (sem_i0, sem_i1)
    sem_rows = (sem_r0, sem_r1)
    sem_out = (sem_o0, sem_o1)
    iota = lax.iota(jnp.int32, L)

    def issue_idx(g, b):
        off = base + g * C
        pltpu.async_copy(ei_hbm.at[pl.ds(off, C)], idx_bufs[b][0],
                         sem_idx[b])
        pltpu.async_copy(ei_hbm.at[pl.ds(N_EDGES + off, C)], idx_bufs[b][1],
                         sem_idx[b])

    def wait_idx(b):
        pltpu.make_async_copy(ei_hbm.at[pl.ds(0, C)], idx_bufs[b][0],
                              sem_idx[b]).wait()
        pltpu.make_async_copy(ei_hbm.at[pl.ds(0, C)], idx_bufs[b][1],
                              sem_idx[b]).wait()

    def issue_gather(b):
        pltpu.async_copy(z_sh.at[idx_bufs[b][0]], rows_bufs[b][0], sem_rows[b])
        pltpu.async_copy(z_sh.at[idx_bufs[b][1]], rows_bufs[b][1], sem_rows[b])

    def wait_gather(b):
        pltpu.make_async_copy(z_hbm.at[pl.ds(0, C)], rows_bufs[b][0],
                              sem_rows[b]).wait()
        pltpu.make_async_copy(z_hbm.at[pl.ds(0, C)], rows_bufs[b][1],
                              sem_rows[b]).wait()

    def issue_out(g, b):
        pltpu.async_copy(outs[b], out_hbm.at[pl.ds(base + g * C, C)],
                         sem_out[b])

    def wait_out(b):
        pltpu.make_async_copy(outs[b], out_hbm.at[pl.ds(0, C)],
                              sem_out[b]).wait()

    def compute(b):
        src_rows, dst_rows = rows_bufs[b]
        out_buf = outs[b]

        @plsc.parallel_loop(0, NGROUP, 1)
        def _(gi):
            gbase = gi * L

            def edge_body(j, acc):
                e = gbase + j
                p = src_rows[e, pl.ds(0, L)] * dst_rows[e, pl.ds(0, L)]
                for kk in range(1, D_FEAT // L):
                    p = p + (src_rows[e, pl.ds(kk * L, L)]
                             * dst_rows[e, pl.ds(kk * L, L)])
                return jnp.where(iota == j, jnp.sum(p), acc)

            acc = lax.fori_loop(0, L, edge_body,
                                jnp.zeros((L,), jnp.float32), unroll=4)
            out = 1.0 / (1.0 + jnp.exp(-acc))
            out_buf[pl.ds(gbase, L)] = out

    def sub_iter(g, b):
        wait_gather(b)

        @pl.when(g + 2 < NCHUNK)
        def _():
            issue_idx(g + 2, b)

        @pl.when(g + 1 < NCHUNK)
        def _():
            wait_idx(1 - b)
            issue_gather(1 - b)

        @pl.when(g >= 2)
        def _():
            wait_out(b)

        compute(b)
        issue_out(g, b)

    @pl.when(sid < NS - 1)
    def _():
        pltpu.sync_copy(z_hbm.at[pl.ds(sid * 632, 632)],
                        z_sh.at[pl.ds(sid * 632, 632)])

    @pl.when(sid == NS - 1)
    def _():
        pltpu.sync_copy(z_hbm.at[pl.ds(9480, 520)],
                        z_sh.at[pl.ds(9480, 520)])

    issue_idx(0, 0)
    issue_idx(1, 1)
    plsc.subcore_barrier()
    wait_idx(0)
    issue_gather(0)

    def pair_body(i, carry):
        sub_iter(2 * i, 0)
        sub_iter(2 * i + 1, 1)
        return carry

    lax.fori_loop(0, NCHUNK // 2, pair_body, 0, unroll=False)
    sub_iter(NCHUNK - 1, 0)

    wait_out(1)
    wait_out(0)


@jax.jit
def _gae_decode(z, edge_index):
    mesh = plsc.VectorSubcoreMesh(core_axis_name="c", subcore_axis_name="s")
    k = pl.kernel(
        _tec_body,
        out_type=jax.ShapeDtypeStruct((N_EDGES,), jnp.float32),
        mesh=mesh,
        compiler_params=pltpu.CompilerParams(needs_layout_passes=False),
        scratch_types=[
            pltpu.VMEM_SHARED((N_NODES, D_FEAT), jnp.float32),
            pltpu.VMEM((C,), jnp.int32),
            pltpu.VMEM((C,), jnp.int32),
            pltpu.VMEM((C,), jnp.int32),
            pltpu.VMEM((C,), jnp.int32),
            pltpu.VMEM((C, D_FEAT), jnp.float32),
            pltpu.VMEM((C, D_FEAT), jnp.float32),
            pltpu.VMEM((C, D_FEAT), jnp.float32),
            pltpu.VMEM((C, D_FEAT), jnp.float32),
            pltpu.VMEM((C,), jnp.float32),
            pltpu.VMEM((C,), jnp.float32),
            pltpu.SemaphoreType.DMA,
            pltpu.SemaphoreType.DMA,
            pltpu.SemaphoreType.DMA,
            pltpu.SemaphoreType.DMA,
            pltpu.SemaphoreType.DMA,
            pltpu.SemaphoreType.DMA,
        ],
    )
    return k(z, edge_index)


def kernel(z, edge_index):
    return _gae_decode(z, edge_index.reshape(-1))
